# baseline (speedup 1.0000x reference)
import functools

import jax
import jax.numpy as jnp
from jax import lax
from jax.experimental import pallas as pl
from jax.experimental.pallas import tpu as pltpu
from jax.experimental.pallas import tpu_sc as plsc

_B, _L, _D = 32, 2048, 512
_K, _E = 32, 256
_NC, _NS, _LANES = 2, 16, 16


def _sc_select_gather_body(feat_hbm, mask_hbm, gath_hbm, sel_hbm,
                           mask_v, sel_v, gidx_v, rows_v, sem):
    b = lax.axis_index("s") * _NC + lax.axis_index("c")
    pltpu.sync_copy(mask_hbm.at[b], mask_v)

    for c in range(_K // _LANES):
        sel_v[pl.ds(c * _LANES, _LANES)] = jnp.full((_LANES,), _L, jnp.int32)

    def chunk_cond(st):
        i, cnt = st
        return jnp.logical_and(i < _L // _LANES, cnt < _K)

    def chunk(st):
        i, cnt = st
        m = mask_v[pl.ds(i * _LANES, _LANES)]
        vmask = m > 0
        ones = vmask.astype(jnp.int32)
        rank = plsc.cumsum(ones) + cnt
        slot = rank - 1
        pos = lax.iota(jnp.int32, _LANES) + i * _LANES
        plsc.store_scatter(sel_v, [slot], pos, mask=vmask & (slot < _K))
        return i + 1, cnt + jnp.sum(ones)

    lax.while_loop(chunk_cond, chunk, (jnp.int32(0), jnp.int32(0)))

    for c in range(_K // _LANES):
        s = sel_v[pl.ds(c * _LANES, _LANES)]
        gidx_v[pl.ds(c * _LANES, _LANES)] = jnp.minimum(s, _L - 1) + b * _L

    pltpu.async_copy(feat_hbm.at[gidx_v], rows_v, sem).wait()
    pltpu.sync_copy(rows_v, gath_hbm.at[b])
    pltpu.sync_copy(sel_v, sel_hbm.at[b])


@functools.cache
def _sc_select_gather():
    return pl.kernel(
        _sc_select_gather_body,
        mesh=plsc.VectorSubcoreMesh(core_axis_name="c", subcore_axis_name="s"),
        compiler_params=pltpu.CompilerParams(needs_layout_passes=False),
        out_type=[
            jax.ShapeDtypeStruct((_B, _K, _D), jnp.float32),
            jax.ShapeDtypeStruct((_B, _K), jnp.int32),
        ],
        scratch_types=[
            pltpu.VMEM((_L,), jnp.int32),
            pltpu.VMEM((_K,), jnp.int32),
            pltpu.VMEM((_K,), jnp.int32),
            pltpu.VMEM((_K, _D), jnp.float32),
            pltpu.SemaphoreType.DMA,
        ],
    )


_RB = 4
_RROWS = _RB * _L


def _reduce_body(maskf_ref, feat_ref, sum_ref, cnt_ref):
    mf = maskf_ref[...]
    x = feat_ref[...]
    sum_ref[...] = jnp.sum((x * mf).reshape(1, _RB, _L, _D), axis=2)
    cnt_ref[...] = jnp.sum(mf.reshape(1, _RB, _L, 1), axis=2)


def _reduce_call(maskf2, feat_flat):
    return pl.pallas_call(
        _reduce_body,
        grid=(_B // _RB,),
        in_specs=[
            pl.BlockSpec((_RROWS, 1), lambda i: (i, 0)),
            pl.BlockSpec((_RROWS, _D), lambda i: (i, 0)),
        ],
        out_specs=[
            pl.BlockSpec((1, _RB, _D), lambda i: (i, 0, 0)),
            pl.BlockSpec((1, _RB, 1), lambda i: (i, 0, 0)),
        ],
        out_shape=[
            jax.ShapeDtypeStruct((_B // _RB, _RB, _D), jnp.float32),
            jax.ShapeDtypeStruct((_B // _RB, _RB, 1), jnp.float32),
        ],
    )(maskf2, feat_flat)


_KC = 8


def _proj_body(ps_ref, cnt_ref, gath_ref, valid_ref, dw_ref, db_ref,
               tw_ref, tb_ref, base_ref, out_ref):
    pooled = ps_ref[...] / jnp.maximum(cnt_ref[...], 1.0)
    g = lax.dot_general(pooled, dw_ref[...], (((1,), (1,)), ((), ())),
                        preferred_element_type=jnp.float32)
    gr = gath_ref[...].reshape(_B * _KC, _D)
    t = lax.dot_general(gr, tw_ref[...], (((1,), (1,)), ((), ())),
                        preferred_element_type=jnp.float32)
    t3 = (t.reshape(_B, _KC, _E) + tb_ref[...]) * valid_ref[...]
    out_ref[...] = (base_ref[...] + db_ref[...]
                    + g.reshape(_B, _KC, _E) + t3)


def _proj_call(psum, cnt, gath3, valid3, delta_W, delta_b3, token_W,
               token_b3, base_tokens):
    return pl.pallas_call(
        _proj_body,
        grid=(_K // _KC,),
        in_specs=[
            pl.BlockSpec((_B, _D), lambda j: (0, 0)),
            pl.BlockSpec((_B, 1), lambda j: (0, 0)),
            pl.BlockSpec((_B, _KC, _D), lambda j: (0, j, 0)),
            pl.BlockSpec((_B, _KC, 1), lambda j: (0, j, 0)),
            pl.BlockSpec((_KC * _E, _D), lambda j: (j, 0)),
            pl.BlockSpec((1, _KC, _E), lambda j: (0, j, 0)),
            pl.BlockSpec((_E, _D), lambda j: (0, 0)),
            pl.BlockSpec((1, 1, _E), lambda j: (0, 0, 0)),
            pl.BlockSpec((1, _KC, _E), lambda j: (0, j, 0)),
        ],
        out_specs=pl.BlockSpec((_B, _KC, _E), lambda j: (0, j, 0)),
        out_shape=jax.ShapeDtypeStruct((_B, _K, _E), jnp.float32),
    )(psum, cnt, gath3, valid3, delta_W, delta_b3, token_W, token_b3,
      base_tokens)


def kernel(text_feat, attention_mask, base_tokens, delta_W, delta_b,
           token_W, token_b):
    feat_flat = text_feat.reshape(_B * _L, _D)
    gathered, sel = _sc_select_gather()(feat_flat, attention_mask)
    maskf2 = (attention_mask > 0).astype(jnp.float32).reshape(_B * _L, 1)
    psum, cnt = _reduce_call(maskf2, feat_flat)
    psum = psum.reshape(_B, _D)
    cnt = cnt.reshape(_B, 1)
    valid3 = (sel < _L).astype(jnp.float32)[:, :, None]
    out = _proj_call(psum, cnt, gathered, valid3,
                     delta_W, delta_b.reshape(1, _K, _E), token_W,
                     token_b.reshape(1, 1, _E), base_tokens)
    return out


# speedup vs baseline: 1.3238x; 1.3238x over previous
import functools

import jax
import jax.numpy as jnp
from jax import lax
from jax.experimental import pallas as pl
from jax.experimental.pallas import tpu as pltpu
from jax.experimental.pallas import tpu_sc as plsc

_B, _L, _D = 32, 2048, 512
_K, _E = 32, 256
_NC, _NS, _LANES = 2, 16, 16


def _sc_select_gather_body(feat_hbm, mask_hbm, gath_hbm, sel_hbm,
                           mask_v, sel_v, gidx_v, rows_v, sem):
    b = lax.axis_index("s") * _NC + lax.axis_index("c")
    pltpu.sync_copy(mask_hbm.at[b], mask_v)

    for c in range(_K // _LANES):
        sel_v[pl.ds(c * _LANES, _LANES)] = jnp.full((_LANES,), _L, jnp.int32)

    def chunk_cond(st):
        i, cnt = st
        return jnp.logical_and(i < _L // _LANES, cnt < _K)

    def chunk(st):
        i, cnt = st
        m = mask_v[pl.ds(i * _LANES, _LANES)]
        vmask = m > 0
        ones = vmask.astype(jnp.int32)
        rank = plsc.cumsum(ones) + cnt
        slot = rank - 1
        pos = lax.iota(jnp.int32, _LANES) + i * _LANES
        plsc.store_scatter(sel_v, [slot], pos, mask=vmask & (slot < _K))
        return i + 1, cnt + jnp.sum(ones)

    lax.while_loop(chunk_cond, chunk, (jnp.int32(0), jnp.int32(0)))

    for c in range(_K // _LANES):
        s = sel_v[pl.ds(c * _LANES, _LANES)]
        gidx_v[pl.ds(c * _LANES, _LANES)] = jnp.minimum(s, _L - 1) + b * _L

    pltpu.async_copy(feat_hbm.at[gidx_v], rows_v, sem).wait()
    pltpu.sync_copy(rows_v, gath_hbm.at[b])
    pltpu.sync_copy(sel_v, sel_hbm.at[b])


@functools.cache
def _sc_select_gather():
    return pl.kernel(
        _sc_select_gather_body,
        mesh=plsc.VectorSubcoreMesh(core_axis_name="c", subcore_axis_name="s"),
        compiler_params=pltpu.CompilerParams(needs_layout_passes=False),
        out_type=[
            jax.ShapeDtypeStruct((_B, _K, _D), jnp.float32),
            jax.ShapeDtypeStruct((_B, _K), jnp.int32),
        ],
        scratch_types=[
            pltpu.VMEM((_L,), jnp.int32),
            pltpu.VMEM((_K,), jnp.int32),
            pltpu.VMEM((_K,), jnp.int32),
            pltpu.VMEM((_K, _D), jnp.float32),
            pltpu.SemaphoreType.DMA,
        ],
    )


_RB = 2
_RROWS = _RB * _L
_NSTEP = _B // _RB
_DWC = _K * _E // _NSTEP


def _mega_body(mask_ref, feat_ref, dw_ref, db_ref, base_ref, out_ref,
               dw_s, ps_s, cnt_s):
    i = pl.program_id(0)
    mf = (mask_ref[0] > 0).astype(jnp.float32)
    x = feat_ref[...].reshape(_RB, _L, _D)
    part = lax.dot_general(mf, x, (((1,), (1,)), ((0,), (0,))),
                           preferred_element_type=jnp.float32)
    ps_s[i] = part
    cnt_s[i] = jnp.sum(mf, axis=1)[:, None]
    dw_s[i] = dw_ref[...]

    @pl.when(i == _NSTEP - 1)
    def _():
        psum = ps_s[...].reshape(_B, _D)
        cnt = cnt_s[...].reshape(_B, 1)
        pooled = psum / jnp.maximum(cnt, 1.0)
        dw = dw_s[...].reshape(_K * _E, _D)
        g = lax.dot_general(pooled, dw, (((1,), (1,)), ((), ())),
                            preferred_element_type=jnp.float32)
        out_ref[...] = base_ref[...] + db_ref[...] + g.reshape(_B, _K, _E)


def _mega_call(mask3, feat_flat, delta_W, delta_b3, base_tokens):
    return pl.pallas_call(
        _mega_body,
        grid=(_NSTEP,),
        in_specs=[
            pl.BlockSpec((1, _RB, _L), lambda i: (i, 0, 0)),
            pl.BlockSpec((_RROWS, _D), lambda i: (i, 0)),
            pl.BlockSpec((_DWC, _D), lambda i: (i, 0)),
            pl.BlockSpec((1, _K, _E), lambda i: (0, 0, 0)),
            pl.BlockSpec((1, _K, _E), lambda i: (0, 0, 0)),
        ],
        out_specs=pl.BlockSpec((_B, _K, _E), lambda i: (0, 0, 0)),
        out_shape=jax.ShapeDtypeStruct((_B, _K, _E), jnp.float32),
        scratch_shapes=[
            pltpu.VMEM((_NSTEP, _DWC, _D), jnp.float32),
            pltpu.VMEM((_NSTEP, _RB, _D), jnp.float32),
            pltpu.VMEM((_NSTEP, _RB, 1), jnp.float32),
        ],
    )(mask3, feat_flat, delta_W, delta_b3, base_tokens)


def _finish_body(og_ref, gath_ref, sel_ref, tw_ref, tb_ref, out_ref):
    gr = gath_ref[...].reshape(_B * _K, _D)
    t = lax.dot_general(gr, tw_ref[...], (((1,), (1,)), ((), ())),
                        preferred_element_type=jnp.float32)
    valid = (sel_ref[...] < _L).astype(jnp.float32)
    t3 = (t.reshape(_B, _K, _E) + tb_ref[...]) * valid
    out_ref[...] = og_ref[...] + t3


def _finish_call(out_global, gath3, sel3, token_W, token_b3):
    return pl.pallas_call(
        _finish_body,
        out_shape=jax.ShapeDtypeStruct((_B, _K, _E), jnp.float32),
    )(out_global, gath3, sel3, token_W, token_b3)


def kernel(text_feat, attention_mask, base_tokens, delta_W, delta_b,
           token_W, token_b):
    feat_flat = text_feat.reshape(_B * _L, _D)
    gathered, sel = _sc_select_gather()(feat_flat, attention_mask)
    mask3 = attention_mask.reshape(_NSTEP, _RB, _L)
    out_global = _mega_call(mask3, feat_flat, delta_W,
                            delta_b.reshape(1, _K, _E), base_tokens)
    out = _finish_call(out_global, gathered, sel[:, :, None],
                       token_W, token_b.reshape(1, 1, _E))
    return out
